# Initial kernel scaffold; baseline (speedup 1.0000x reference)
import jax
import jax.numpy as jnp
from jax.experimental import pallas as pl


def kernel(x, weight, a_param):
    raise NotImplementedError("write your pallas kernel here")


# speedup vs baseline: 10.7907x; 10.7907x over previous
import jax
import jax.numpy as jnp
from jax.experimental import pallas as pl
from jax.experimental.pallas import tpu as pltpu

N = 4096
D = 512
K = 128
PBLK = 512
BLK = 256

KEY_LO = -1073741825
KEY_HI = 1073741824
_MASK = 0x7FFFFFFF


def _prep_body(x_ref, w_ref, a_ref, out_ref):
    x = x_ref[...]
    fw = jax.nn.softmax(a_ref[...], axis=-1)
    h = jax.lax.dot_general(x, w_ref[...], (((1,), (1,)), ((), ())),
                            preferred_element_type=jnp.float32,
                            precision=jax.lax.Precision.HIGHEST)
    out_ref[...] = jnp.clip(h * fw, -1.0, 1.0)


def _att_body(xb_ref, x_ref, out_ref, y_ref):
    xb = xb_ref[...]
    xf = x_ref[...]
    s = jax.lax.dot_general(xb, xf, (((1,), (1,)), ((), ())),
                            preferred_element_type=jnp.float32,
                            precision=jax.lax.Precision.DEFAULT)
    nsq_row = jnp.sum(xb * xb, axis=1, keepdims=True)
    nsq_col = jax.lax.dot_general(jnp.ones((1, D), jnp.float32), xf * xf,
                                  (((1,), (1,)), ((), ())),
                                  preferred_element_type=jnp.float32,
                                  precision=jax.lax.Precision.HIGHEST)
    att = s / (jnp.sqrt(nsq_row) * jnp.sqrt(nsq_col))

    bits = jax.lax.bitcast_convert_type(att, jnp.int32)
    keys = jnp.where(bits >= 0, bits, bits ^ _MASK)

    lo = jnp.full((BLK, 1), KEY_LO, jnp.int32)
    hi = jnp.full((BLK, 1), KEY_HI, jnp.int32)

    def body(_, carry):
        lo, hi = carry
        mid = (lo >> 1) + (hi >> 1) + (lo & hi & 1)
        cnt = jnp.sum((keys >= mid).astype(jnp.float32), axis=1, keepdims=True)
        ge = cnt >= float(K)
        return jnp.where(ge, mid, lo), jnp.where(ge, hi, mid)

    lo, hi = jax.lax.fori_loop(0, 32, body, (lo, hi))

    m = jnp.max(att, axis=1, keepdims=True)
    p = jnp.where(keys >= lo, jnp.exp(att - m), 0.0)
    z = jnp.sum(p, axis=1, keepdims=True)
    y = jax.lax.dot_general(p, out_ref[...], (((1,), (0,)), ((), ())),
                            preferred_element_type=jnp.float32,
                            precision=jax.lax.Precision.HIGHEST)
    y_ref[...] = y / z


def kernel(x, weight, a_param):
    a2 = a_param.reshape(1, D)
    out = pl.pallas_call(
        _prep_body,
        grid=(N // PBLK,),
        in_specs=[pl.BlockSpec((PBLK, D), lambda i: (i, 0)),
                  pl.BlockSpec((D, D), lambda i: (0, 0)),
                  pl.BlockSpec((1, D), lambda i: (0, 0))],
        out_specs=pl.BlockSpec((PBLK, D), lambda i: (i, 0)),
        out_shape=jax.ShapeDtypeStruct((N, D), jnp.float32),
        compiler_params=pltpu.CompilerParams(
            dimension_semantics=("arbitrary",)),
    )(x, weight, a2)
    y = pl.pallas_call(
        _att_body,
        grid=(N // BLK,),
        in_specs=[pl.BlockSpec((BLK, D), lambda i: (i, 0)),
                  pl.BlockSpec((N, D), lambda i: (0, 0)),
                  pl.BlockSpec((N, D), lambda i: (0, 0))],
        out_specs=pl.BlockSpec((BLK, D), lambda i: (i, 0)),
        out_shape=jax.ShapeDtypeStruct((N, D), jnp.float32),
        compiler_params=pltpu.CompilerParams(
            dimension_semantics=("arbitrary",)),
    )(x, x, out)
    return y


# speedup vs baseline: 13.1368x; 1.2174x over previous
import jax
import jax.numpy as jnp
from jax.experimental import pallas as pl
from jax.experimental.pallas import tpu as pltpu

N = 4096
D = 512
K = 128
PBLK = 512
BLK = 256

KEY_LO = -1073741825
KEY_HI = 1073741824
_MASK = 0x7FFFFFFF


def _prep_body(x_ref, w_ref, a_ref, out_ref):
    x = x_ref[...]
    fw = jax.nn.softmax(a_ref[...], axis=-1)
    h = jax.lax.dot_general(x, w_ref[...], (((1,), (1,)), ((), ())),
                            preferred_element_type=jnp.float32,
                            precision=jax.lax.Precision.DEFAULT)
    out_ref[...] = jnp.clip(h * fw, -1.0, 1.0)


def _att_body(xb_ref, x_ref, out_ref, y_ref):
    xb = xb_ref[...]
    xf = x_ref[...]
    s = jax.lax.dot_general(xb, xf, (((1,), (1,)), ((), ())),
                            preferred_element_type=jnp.float32,
                            precision=jax.lax.Precision.DEFAULT)
    nsq_row = jnp.sum(xb * xb, axis=1, keepdims=True)
    nsq_col = jax.lax.dot_general(jnp.ones((1, D), jnp.float32), xf * xf,
                                  (((1,), (1,)), ((), ())),
                                  preferred_element_type=jnp.float32,
                                  precision=jax.lax.Precision.HIGHEST)
    att = s / (jnp.sqrt(nsq_row) * jnp.sqrt(nsq_col))

    bits = jax.lax.bitcast_convert_type(att, jnp.int32)
    keys = jnp.where(bits >= 0, bits, bits ^ _MASK)

    lo = jnp.full((BLK, 1), KEY_LO, jnp.int32)
    hi = jnp.full((BLK, 1), KEY_HI, jnp.int32)

    def body(_, carry):
        lo, hi = carry
        mid = (lo >> 1) + (hi >> 1) + (lo & hi & 1)
        cnt = jnp.sum((keys >= mid).astype(jnp.float32), axis=1, keepdims=True)
        ge = cnt >= float(K)
        return jnp.where(ge, mid, lo), jnp.where(ge, hi, mid)

    lo, hi = jax.lax.fori_loop(0, 32, body, (lo, hi))

    m = jnp.max(att, axis=1, keepdims=True)
    p = jnp.where(keys >= lo, jnp.exp(att - m), 0.0)
    z = jnp.sum(p, axis=1, keepdims=True)
    y = jax.lax.dot_general(p, out_ref[...], (((1,), (0,)), ((), ())),
                            preferred_element_type=jnp.float32,
                            precision=jax.lax.Precision.DEFAULT)
    y_ref[...] = y / z


def kernel(x, weight, a_param):
    a2 = a_param.reshape(1, D)
    out = pl.pallas_call(
        _prep_body,
        grid=(N // PBLK,),
        in_specs=[pl.BlockSpec((PBLK, D), lambda i: (i, 0)),
                  pl.BlockSpec((D, D), lambda i: (0, 0)),
                  pl.BlockSpec((1, D), lambda i: (0, 0))],
        out_specs=pl.BlockSpec((PBLK, D), lambda i: (i, 0)),
        out_shape=jax.ShapeDtypeStruct((N, D), jnp.float32),
        compiler_params=pltpu.CompilerParams(
            dimension_semantics=("arbitrary",)),
    )(x, weight, a2)
    y = pl.pallas_call(
        _att_body,
        grid=(N // BLK,),
        in_specs=[pl.BlockSpec((BLK, D), lambda i: (i, 0)),
                  pl.BlockSpec((N, D), lambda i: (0, 0)),
                  pl.BlockSpec((N, D), lambda i: (0, 0))],
        out_specs=pl.BlockSpec((BLK, D), lambda i: (i, 0)),
        out_shape=jax.ShapeDtypeStruct((N, D), jnp.float32),
        compiler_params=pltpu.CompilerParams(
            dimension_semantics=("arbitrary",)),
    )(x, x, out)
    return y


# speedup vs baseline: 18.0390x; 1.3732x over previous
import jax
import jax.numpy as jnp
from jax.experimental import pallas as pl
from jax.experimental.pallas import tpu as pltpu

N = 4096
D = 512
K = 128
PBLK = 512
BLK = 256
NG = 128

_MASK = 0x7FFFFFFF
_GMIN = -2147483647


def _prep_body(x_ref, w_ref, a_ref, out_ref):
    x = x_ref[...]
    fw = jax.nn.softmax(a_ref[...], axis=-1)
    h = jax.lax.dot_general(x, w_ref[...], (((1,), (1,)), ((), ())),
                            preferred_element_type=jnp.float32,
                            precision=jax.lax.Precision.DEFAULT)
    out_ref[...] = jnp.clip(h * fw, -1.0, 1.0)


def _att_body(xb_ref, x_ref, out_ref, y_ref, ncs_ref):
    xf = x_ref[...]

    @pl.when(pl.program_id(0) == 0)
    def _():
        nsq = jax.lax.dot_general(jnp.ones((1, D), jnp.float32), xf * xf,
                                  (((1,), (1,)), ((), ())),
                                  preferred_element_type=jnp.float32,
                                  precision=jax.lax.Precision.HIGHEST)
        ncs_ref[...] = jnp.sqrt(nsq)

    xb = xb_ref[...]
    s = jax.lax.dot_general(xb, xf, (((1,), (1,)), ((), ())),
                            preferred_element_type=jnp.float32,
                            precision=jax.lax.Precision.DEFAULT)
    nr = jnp.sqrt(jnp.sum(xb * xb, axis=1, keepdims=True))
    att = s / (nr * ncs_ref[...])

    bits = jax.lax.bitcast_convert_type(att, jnp.int32)
    keys = jnp.where(bits >= 0, bits, bits ^ _MASK)

    def count_ge(t):
        return jnp.sum((keys >= t).astype(jnp.float32), axis=1, keepdims=True)

    grp = jnp.max(keys.reshape(BLK, N // NG, NG), axis=1)
    g1 = jnp.max(grp, axis=1, keepdims=True)
    lo = jnp.min(grp, axis=1, keepdims=True)
    hi = g1 + 1
    clo = count_ge(lo)
    chi = jnp.zeros_like(clo)

    g = jnp.where(grp >= g1, _GMIN, grp)
    g2 = jnp.max(g, axis=1, keepdims=True)
    g = jnp.where(g >= g2, _GMIN, g)
    g3 = jnp.max(g, axis=1, keepdims=True)
    g = jnp.where(g >= g3, _GMIN, g)
    g4 = jnp.max(g, axis=1, keepdims=True)
    probe = jnp.clip(g4, lo + 1, hi - 1)
    open_ = (hi - lo) > 1
    c4 = count_ge(probe)
    ge = c4 >= float(K)
    clo = jnp.where(open_ & ge, c4, clo)
    chi = jnp.where(open_ & ~ge, c4, chi)
    lo = jnp.where(open_ & ge, probe, lo)
    hi = jnp.where(open_ & ~ge, probe, hi)

    def _done(lo, hi, clo):
        return (clo == float(K)) | ((hi - lo) <= 1)

    def cond(c):
        lo, hi, clo, _, _ = c
        return jnp.any(~_done(lo, hi, clo))

    def body(c):
        lo, hi, clo, chi, it = c
        span = (hi - lo).astype(jnp.float32)
        frac = (clo - float(K)) / jnp.maximum(clo - chi, 1.0)
        mid_fp = lo + jnp.clip((span * frac).astype(jnp.int32), 1, hi - 1 - lo)
        mid_bi = (lo >> 1) + (hi >> 1) + (lo & hi & 1)
        mid = jnp.where(it % 2 == 0, mid_fp, mid_bi)
        cnt = count_ge(mid)
        ge = cnt >= float(K)
        act = ~_done(lo, hi, clo)
        upd_lo = act & ge
        upd_hi = act & ~ge
        return (jnp.where(upd_lo, mid, lo), jnp.where(upd_hi, mid, hi),
                jnp.where(upd_lo, cnt, clo), jnp.where(upd_hi, cnt, chi),
                it + 1)

    lo, hi, clo, chi, _ = jax.lax.while_loop(
        cond, body, (lo, hi, clo, chi, jnp.int32(0)))

    mb = jnp.where(g1 >= 0, g1, g1 ^ _MASK)
    m = jax.lax.bitcast_convert_type(mb, jnp.float32)
    p = jnp.where(keys >= lo, jnp.exp(att - m), 0.0)
    z = jnp.sum(p, axis=1, keepdims=True)
    y = jax.lax.dot_general(p, out_ref[...], (((1,), (0,)), ((), ())),
                            preferred_element_type=jnp.float32,
                            precision=jax.lax.Precision.DEFAULT)
    y_ref[...] = y / z


def kernel(x, weight, a_param):
    a2 = a_param.reshape(1, D)
    out = pl.pallas_call(
        _prep_body,
        grid=(N // PBLK,),
        in_specs=[pl.BlockSpec((PBLK, D), lambda i: (i, 0)),
                  pl.BlockSpec((D, D), lambda i: (0, 0)),
                  pl.BlockSpec((1, D), lambda i: (0, 0))],
        out_specs=pl.BlockSpec((PBLK, D), lambda i: (i, 0)),
        out_shape=jax.ShapeDtypeStruct((N, D), jnp.float32),
        compiler_params=pltpu.CompilerParams(
            dimension_semantics=("arbitrary",)),
    )(x, weight, a2)
    y = pl.pallas_call(
        _att_body,
        grid=(N // BLK,),
        in_specs=[pl.BlockSpec((BLK, D), lambda i: (i, 0)),
                  pl.BlockSpec((N, D), lambda i: (0, 0)),
                  pl.BlockSpec((N, D), lambda i: (0, 0))],
        out_specs=pl.BlockSpec((BLK, D), lambda i: (i, 0)),
        out_shape=jax.ShapeDtypeStruct((N, D), jnp.float32),
        scratch_shapes=[pltpu.VMEM((1, N), jnp.float32)],
        compiler_params=pltpu.CompilerParams(
            dimension_semantics=("arbitrary",)),
    )(x, x, out)
    return y


# speedup vs baseline: 18.2282x; 1.0105x over previous
import jax
import jax.numpy as jnp
from jax.experimental import pallas as pl
from jax.experimental.pallas import tpu as pltpu

N = 4096
D = 512
K = 128
PBLK = 512
BLK = 512
NG = 128

_MASK = 0x7FFFFFFF
_GMIN = -2147483647


def _prep_body(x_ref, w_ref, a_ref, out_ref):
    x = x_ref[...]
    fw = jax.nn.softmax(a_ref[...], axis=-1)
    h = jax.lax.dot_general(x, w_ref[...], (((1,), (1,)), ((), ())),
                            preferred_element_type=jnp.float32,
                            precision=jax.lax.Precision.DEFAULT)
    out_ref[...] = jnp.clip(h * fw, -1.0, 1.0)


def _att_body(xb_ref, x_ref, out_ref, y_ref, ncs_ref):
    xf = x_ref[...]

    @pl.when(pl.program_id(0) == 0)
    def _():
        nsq = jax.lax.dot_general(jnp.ones((1, D), jnp.float32), xf * xf,
                                  (((1,), (1,)), ((), ())),
                                  preferred_element_type=jnp.float32,
                                  precision=jax.lax.Precision.HIGHEST)
        ncs_ref[...] = jnp.sqrt(nsq)

    xb = xb_ref[...]
    s = jax.lax.dot_general(xb, xf, (((1,), (1,)), ((), ())),
                            preferred_element_type=jnp.float32,
                            precision=jax.lax.Precision.DEFAULT)
    nr = jnp.sqrt(jnp.sum(xb * xb, axis=1, keepdims=True))
    att = s / (nr * ncs_ref[...])

    bits = jax.lax.bitcast_convert_type(att, jnp.int32)
    keys = jnp.where(bits >= 0, bits, bits ^ _MASK)

    def count_ge(t):
        return jnp.sum((keys >= t).astype(jnp.float32), axis=1, keepdims=True)

    grp = jnp.max(keys.reshape(BLK, N // NG, NG), axis=1)
    g1 = jnp.max(grp, axis=1, keepdims=True)
    lo = jnp.min(grp, axis=1, keepdims=True)
    hi = g1 + 1
    clo = count_ge(lo)
    chi = jnp.zeros_like(clo)

    g = jnp.where(grp >= g1, _GMIN, grp)
    g2 = jnp.max(g, axis=1, keepdims=True)
    g = jnp.where(g >= g2, _GMIN, g)
    g3 = jnp.max(g, axis=1, keepdims=True)
    g = jnp.where(g >= g3, _GMIN, g)
    g4 = jnp.max(g, axis=1, keepdims=True)
    probe = jnp.clip(g4, lo + 1, hi - 1)
    open_ = (hi - lo) > 1
    c4 = count_ge(probe)
    ge = c4 >= float(K)
    clo = jnp.where(open_ & ge, c4, clo)
    chi = jnp.where(open_ & ~ge, c4, chi)
    lo = jnp.where(open_ & ge, probe, lo)
    hi = jnp.where(open_ & ~ge, probe, hi)

    def _done(lo, hi, clo):
        return (clo == float(K)) | ((hi - lo) <= 1)

    def cond(c):
        lo, hi, clo, _, _ = c
        return jnp.any(~_done(lo, hi, clo))

    def body(c):
        lo, hi, clo, chi, it = c
        span = (hi - lo).astype(jnp.float32)
        frac = (clo - float(K)) / jnp.maximum(clo - chi, 1.0)
        mid_fp = lo + jnp.clip((span * frac).astype(jnp.int32), 1, hi - 1 - lo)
        mid_bi = (lo >> 1) + (hi >> 1) + (lo & hi & 1)
        mid = jnp.where(it % 2 == 0, mid_fp, mid_bi)
        cnt = count_ge(mid)
        ge = cnt >= float(K)
        act = ~_done(lo, hi, clo)
        upd_lo = act & ge
        upd_hi = act & ~ge
        return (jnp.where(upd_lo, mid, lo), jnp.where(upd_hi, mid, hi),
                jnp.where(upd_lo, cnt, clo), jnp.where(upd_hi, cnt, chi),
                it + 1)

    lo, hi, clo, chi, _ = jax.lax.while_loop(
        cond, body, (lo, hi, clo, chi, jnp.int32(0)))

    mb = jnp.where(g1 >= 0, g1, g1 ^ _MASK)
    m = jax.lax.bitcast_convert_type(mb, jnp.float32)
    p = jnp.where(keys >= lo, jnp.exp(att - m), 0.0)
    z = jnp.sum(p, axis=1, keepdims=True)
    y = jax.lax.dot_general(p, out_ref[...], (((1,), (0,)), ((), ())),
                            preferred_element_type=jnp.float32,
                            precision=jax.lax.Precision.DEFAULT)
    y_ref[...] = y / z


def kernel(x, weight, a_param):
    a2 = a_param.reshape(1, D)
    out = pl.pallas_call(
        _prep_body,
        grid=(N // PBLK,),
        in_specs=[pl.BlockSpec((PBLK, D), lambda i: (i, 0)),
                  pl.BlockSpec((D, D), lambda i: (0, 0)),
                  pl.BlockSpec((1, D), lambda i: (0, 0))],
        out_specs=pl.BlockSpec((PBLK, D), lambda i: (i, 0)),
        out_shape=jax.ShapeDtypeStruct((N, D), jnp.float32),
        compiler_params=pltpu.CompilerParams(
            dimension_semantics=("arbitrary",)),
    )(x, weight, a2)
    y = pl.pallas_call(
        _att_body,
        grid=(N // BLK,),
        in_specs=[pl.BlockSpec((BLK, D), lambda i: (i, 0)),
                  pl.BlockSpec((N, D), lambda i: (0, 0)),
                  pl.BlockSpec((N, D), lambda i: (0, 0))],
        out_specs=pl.BlockSpec((BLK, D), lambda i: (i, 0)),
        out_shape=jax.ShapeDtypeStruct((N, D), jnp.float32),
        scratch_shapes=[pltpu.VMEM((1, N), jnp.float32)],
        compiler_params=pltpu.CompilerParams(
            dimension_semantics=("arbitrary",)),
    )(x, x, out)
    return y


# speedup vs baseline: 20.3567x; 1.1168x over previous
import jax
import jax.numpy as jnp
from jax.experimental import pallas as pl
from jax.experimental.pallas import tpu as pltpu

N = 4096
D = 512
K = 128
PBLK = 512
BLK = 512
NG = 128

_MASK = 0x7FFFFFFF
_GMIN = -2147483647


def _prep_body(x_ref, w_ref, a_ref, out_ref):
    x = x_ref[...]
    fw = jax.nn.softmax(a_ref[...], axis=-1)
    h = jax.lax.dot_general(x, w_ref[...], (((1,), (1,)), ((), ())),
                            preferred_element_type=jnp.float32,
                            precision=jax.lax.Precision.DEFAULT)
    out_ref[...] = jnp.clip(h * fw, -1.0, 1.0)


def _att_body(xb_ref, x_ref, out_ref, y_ref, ncs_ref):
    xf = x_ref[...]

    @pl.when(pl.program_id(0) == 0)
    def _():
        nsq = jax.lax.dot_general(jnp.ones((1, D), jnp.float32), xf * xf,
                                  (((1,), (1,)), ((), ())),
                                  preferred_element_type=jnp.float32,
                                  precision=jax.lax.Precision.HIGHEST)
        ncs_ref[...] = jnp.sqrt(nsq)

    xb = xb_ref[...]
    s = jax.lax.dot_general(xb, xf, (((1,), (1,)), ((), ())),
                            preferred_element_type=jnp.float32,
                            precision=jax.lax.Precision.DEFAULT)
    nr = jnp.sqrt(jnp.sum(xb * xb, axis=1, keepdims=True))
    att = s / (nr * ncs_ref[...])

    bits = jax.lax.bitcast_convert_type(att, jnp.int32)
    keys = jnp.where(bits >= 0, bits, bits ^ _MASK)

    def count_ge(t):
        return jnp.sum((keys >= t).astype(jnp.float32), axis=1, keepdims=True)

    grp = keys[:, 0:NG]
    for c in range(1, N // NG):
        grp = jnp.maximum(grp, keys[:, c * NG:(c + 1) * NG])
    g1 = jnp.max(grp, axis=1, keepdims=True)
    lo = jnp.min(grp, axis=1, keepdims=True)
    hi = g1 + 1
    clo = count_ge(lo)
    chi = jnp.zeros_like(clo)

    g = jnp.where(grp >= g1, _GMIN, grp)
    g2 = jnp.max(g, axis=1, keepdims=True)
    g = jnp.where(g >= g2, _GMIN, g)
    g3 = jnp.max(g, axis=1, keepdims=True)
    g = jnp.where(g >= g3, _GMIN, g)
    g4 = jnp.max(g, axis=1, keepdims=True)
    probe = jnp.clip(g4, lo + 1, hi - 1)
    open_ = (hi - lo) > 1
    c4 = count_ge(probe)
    ge = c4 >= float(K)
    clo = jnp.where(open_ & ge, c4, clo)
    chi = jnp.where(open_ & ~ge, c4, chi)
    lo = jnp.where(open_ & ge, probe, lo)
    hi = jnp.where(open_ & ~ge, probe, hi)

    def _done(lo, hi, clo):
        return (clo == float(K)) | ((hi - lo) <= 128)

    def cond(c):
        lo, hi, clo, _, _ = c
        return jnp.any(~_done(lo, hi, clo))

    def body(c):
        lo, hi, clo, chi, it = c
        span = (hi - lo).astype(jnp.float32)
        frac = (clo - float(K)) / jnp.maximum(clo - chi, 1.0)
        mid_fp = lo + jnp.clip((span * frac).astype(jnp.int32), 1, hi - 1 - lo)
        mid_bi = (lo >> 1) + (hi >> 1) + (lo & hi & 1)
        mid = jnp.where(it % 2 == 0, mid_fp, mid_bi)
        cnt = count_ge(mid)
        ge = cnt >= float(K)
        act = ~_done(lo, hi, clo)
        upd_lo = act & ge
        upd_hi = act & ~ge
        return (jnp.where(upd_lo, mid, lo), jnp.where(upd_hi, mid, hi),
                jnp.where(upd_lo, cnt, clo), jnp.where(upd_hi, cnt, chi),
                it + 1)

    lo, hi, clo, chi, _ = jax.lax.while_loop(
        cond, body, (lo, hi, clo, chi, jnp.int32(0)))

    p = jnp.where(keys >= lo, jnp.exp(att), 0.0)
    z = jnp.sum(p, axis=1, keepdims=True)
    y = jax.lax.dot_general(p.astype(jnp.bfloat16), out_ref[...],
                            (((1,), (0,)), ((), ())),
                            preferred_element_type=jnp.float32,
                            precision=jax.lax.Precision.DEFAULT)
    y_ref[...] = y / z


def kernel(x, weight, a_param):
    a2 = a_param.reshape(1, D)
    out = pl.pallas_call(
        _prep_body,
        grid=(N // PBLK,),
        in_specs=[pl.BlockSpec((PBLK, D), lambda i: (i, 0)),
                  pl.BlockSpec((D, D), lambda i: (0, 0)),
                  pl.BlockSpec((1, D), lambda i: (0, 0))],
        out_specs=pl.BlockSpec((PBLK, D), lambda i: (i, 0)),
        out_shape=jax.ShapeDtypeStruct((N, D), jnp.float32),
        compiler_params=pltpu.CompilerParams(
            dimension_semantics=("arbitrary",)),
    )(x, weight, a2)
    y = pl.pallas_call(
        _att_body,
        grid=(N // BLK,),
        in_specs=[pl.BlockSpec((BLK, D), lambda i: (i, 0)),
                  pl.BlockSpec((N, D), lambda i: (0, 0)),
                  pl.BlockSpec((N, D), lambda i: (0, 0))],
        out_specs=pl.BlockSpec((BLK, D), lambda i: (i, 0)),
        out_shape=jax.ShapeDtypeStruct((N, D), jnp.float32),
        scratch_shapes=[pltpu.VMEM((1, N), jnp.float32)],
        compiler_params=pltpu.CompilerParams(
            dimension_semantics=("arbitrary",)),
    )(x, x, out)
    return y
